# Initial kernel scaffold; baseline (speedup 1.0000x reference)
import jax
import jax.numpy as jnp
from jax.experimental import pallas as pl


def kernel(x, edge_index, edge_weight, W, b):
    raise NotImplementedError("write your pallas kernel here")


# speedup vs baseline: 17.3863x; 17.3863x over previous
import functools

import jax
import jax.numpy as jnp
from jax import lax
from jax.experimental import pallas as pl
from jax.experimental.pallas import tpu as pltpu
from jax.experimental.pallas import tpu_sc as plsc

N = 10000
E = 320000
D = 128

NC = 2
NS = 16
NW = NC * NS
L = 16

CHUNK = 128
EPT = -(-E // NW)
NCHUNK = -(-EPT // CHUNK)
EPAD = NW * NCHUNK * CHUNK

_mesh = plsc.VectorSubcoreMesh(core_axis_name="c", subcore_axis_name="s")


@functools.partial(
    pl.kernel,
    out_type=jax.ShapeDtypeStruct((NW, N), jnp.float32),
    mesh=_mesh,
    compiler_params=pltpu.CompilerParams(needs_layout_passes=False),
    scratch_types=[
        pltpu.VMEM((NCHUNK, CHUNK), jnp.int32),
        pltpu.VMEM((NCHUNK, CHUNK), jnp.float32),
        pltpu.VMEM((N,), jnp.float32),
    ],
)
def _deg_kernel(col_hbm, ew_hbm, out_hbm, col_v, ew_v, deg_v):
    c = lax.axis_index("c")
    s = lax.axis_index("s")
    wid = s * NC + c
    pltpu.sync_copy(col_hbm.at[wid], col_v)
    pltpu.sync_copy(ew_hbm.at[wid], ew_v)

    zeros = jnp.zeros((L,), jnp.float32)

    def zbody(i, carry):
        deg_v[pl.ds(i * L, L)] = zeros
        return carry

    lax.fori_loop(0, N // L, zbody, 0)

    def ebody(j, carry):
        for g in range(CHUNK // L):
            cols = col_v[j, pl.ds(g * L, L)]
            ews = ew_v[j, pl.ds(g * L, L)]
            plsc.addupdate_scatter(deg_v, [cols], ews)
        return carry

    lax.fori_loop(0, NCHUNK, ebody, 0)
    pltpu.sync_copy(deg_v, out_hbm.at[wid])


def _prep_body(x_ref, w_ref, degp_ref, g_ref, dis_ref):
    deg = jnp.sum(degp_ref[...], axis=0) + 1.0
    dis = jnp.where(deg > 0, lax.rsqrt(deg), 0.0)
    h = jnp.dot(x_ref[...], w_ref[...], preferred_element_type=jnp.float32)
    g_ref[...] = h * dis[:, None]
    dis_ref[...] = dis[:, None]


_prep_call = pl.pallas_call(
    _prep_body,
    out_shape=(
        jax.ShapeDtypeStruct((N, D), jnp.float32),
        jax.ShapeDtypeStruct((N, 1), jnp.float32),
    ),
)


@functools.partial(
    pl.kernel,
    out_type=jax.ShapeDtypeStruct((NC, N, D), jnp.float32),
    mesh=_mesh,
    compiler_params=pltpu.CompilerParams(needs_layout_passes=False),
    scratch_types=[
        pltpu.VMEM((NCHUNK, CHUNK), jnp.int32),
        pltpu.VMEM((NCHUNK, CHUNK), jnp.int32),
        pltpu.VMEM((NCHUNK, CHUNK), jnp.float32),
        pltpu.VMEM((CHUNK, D), jnp.float32),
        pltpu.VMEM_SHARED((N, D), jnp.float32),
        pltpu.SemaphoreType.DMA,
    ],
)
def _scat_kernel(g_hbm, row_hbm, col_hbm, ew_hbm, z_hbm, out_hbm,
                 row_v, col_v, ew_v, buf, acc, sem):
    c = lax.axis_index("c")
    s = lax.axis_index("s")
    wid = s * NC + c
    pltpu.sync_copy(row_hbm.at[wid], row_v)
    pltpu.sync_copy(col_hbm.at[wid], col_v)
    pltpu.sync_copy(ew_hbm.at[wid], ew_v)

    rp = (N // NS) // 8 * 8
    tail = N - NS * rp
    pltpu.sync_copy(z_hbm.at[pl.ds(s * rp, rp)], acc.at[pl.ds(s * rp, rp)])

    @pl.when(s == NS - 1)
    def _zero_tail():
        pltpu.sync_copy(z_hbm.at[pl.ds(NS * rp, tail)],
                        acc.at[pl.ds(NS * rp, tail)])

    plsc.subcore_barrier()

    def chunk_body(j, carry):
        pltpu.async_copy(g_hbm.at[row_v.at[j]], buf, sem).wait()

        def grp_body(g, inner):
            ewv = ew_v[j, pl.ds(g * L, L)]
            for r in range(L):
                w = ewv[r]
                for dd in range(D // L):
                    sl = buf[g * L + r, pl.ds(dd * L, L)]
                    buf[g * L + r, pl.ds(dd * L, L)] = sl * w
            return inner

        lax.fori_loop(0, CHUNK // L, grp_body, 0)
        pltpu.sync_copy(buf, acc.at[col_v.at[j]], add=True)
        return carry

    lax.fori_loop(0, NCHUNK, chunk_body, 0)
    plsc.subcore_barrier()
    pltpu.sync_copy(acc.at[pl.ds(s * rp, rp)], out_hbm.at[c, pl.ds(s * rp, rp)])

    @pl.when(s == NS - 1)
    def _dump_tail():
        pltpu.sync_copy(acc.at[pl.ds(NS * rp, tail)],
                        out_hbm.at[c, pl.ds(NS * rp, tail)])


def _comb_body(p_ref, g_ref, dis_ref, b_ref, o_ref):
    o_ref[...] = dis_ref[...] * (p_ref[0] + p_ref[1] + g_ref[...]) + b_ref[...]


_comb_call = pl.pallas_call(
    _comb_body,
    out_shape=jax.ShapeDtypeStruct((N, D), jnp.float32),
)


def kernel(x, edge_index, edge_weight, W, b):
    row = edge_index[0]
    col = edge_index[1]
    pad = EPAD - E
    rowp = jnp.pad(row, (0, pad)).reshape(NW, NCHUNK, CHUNK)
    colp = jnp.pad(col, (0, pad)).reshape(NW, NCHUNK, CHUNK)
    ewp = jnp.pad(edge_weight, (0, pad)).reshape(NW, NCHUNK, CHUNK)
    zeros = jnp.zeros((N, D), jnp.float32)

    degp = _deg_kernel(colp, ewp)
    g, dis = _prep_call(x, W, degp)
    p = _scat_kernel(g, rowp, colp, ewp, zeros)
    return _comb_call(p, g, dis, b)
